# Initial kernel scaffold; baseline (speedup 1.0000x reference)
import jax
import jax.numpy as jnp
from jax.experimental import pallas as pl


def kernel(predicts, targets):
    raise NotImplementedError("write your pallas kernel here")


# speedup vs baseline: 4.0013x; 4.0013x over previous
import jax
import jax.numpy as jnp
from jax import lax
from jax.experimental import pallas as pl
from jax.experimental.pallas import tpu as pltpu
from jax.experimental.pallas import tpu_sc as plsc

N_CLS = 10
B = 8
N = 32768
NC = 2
NS = 16
BPC = B // NC
CPB = NS // BPC
CHUNK = N // CPB
ROWS_BUF = 2048
WORDS_BUF = ROWS_BUF * N_CLS
NDMA = CHUNK // ROWS_BUF
HBINS = 16 * 128
INF = float("inf")

_mesh = plsc.VectorSubcoreMesh(
    core_axis_name="c", subcore_axis_name="s", num_cores=NC, num_subcores=NS
)


_OUT_TYPE = [
    jax.ShapeDtypeStruct((B, N * N_CLS), jnp.int32),
    jax.ShapeDtypeStruct((NC, 16), jnp.int32),
]
_SCRATCH_TYPES = [
    pltpu.VMEM((CHUNK,), jnp.int32),
    pltpu.VMEM((CHUNK,), jnp.int32),
    pltpu.VMEM((HBINS,), jnp.int32),
    pltpu.VMEM((HBINS,), jnp.int32),
    pltpu.VMEM((11, 16), jnp.float32),
    pltpu.VMEM((16,), jnp.float32),
    pltpu.VMEM((16,), jnp.int32),
    pltpu.VMEM((WORDS_BUF,), jnp.int32),
    pltpu.VMEM_SHARED((NS, HBINS), jnp.int32),
    pltpu.VMEM_SHARED((BPC, 128), jnp.int32),
]


def _miou_body(predicts_hbm, targets_hbm, out_hbm, tin_hbm,
             t_v, p_v, hist_v, tmp_v, cost_v, u_v, ans_v, rep_v,
             sh_hist, sh_res):
    c = lax.axis_index("c")
    s = lax.axis_index("s")
    b_local = s // CPB
    chunk = s % CPB
    b = c * BPC + b_local
    iota = lax.broadcasted_iota(jnp.int32, (16,), 0)
    zeros_i = jnp.zeros((16,), jnp.int32)
    ones_i = jnp.ones((16,), jnp.int32)

    pltpu.sync_copy(targets_hbm.at[b, pl.ds(chunk * CHUNK, CHUNK)], t_v)
    pltpu.sync_copy(predicts_hbm.at[b, pl.ds(chunk * CHUNK, CHUNK)], p_v)

    def zero_body(k, _):
        hist_v[pl.ds(k * 16, 16)] = zeros_i
        return _

    lax.fori_loop(0, HBINS // 16, zero_body, 0)

    bank = iota * 128

    def cnt_body(n, _):
        t = t_v[pl.ds(n * 16, 16)]
        p = p_v[pl.ds(n * 16, 16)]
        idx = bank + t * N_CLS + p
        plsc.addupdate_scatter(hist_v, [idx], ones_i)
        return _

    lax.fori_loop(0, CHUNK // 16, cnt_body, 0)

    pltpu.sync_copy(hist_v, sh_hist.at[s])
    plsc.subcore_barrier()

    @pl.when(chunk == 0)
    def _owner():
        for r in range(1, CPB):
            pltpu.sync_copy(sh_hist.at[s + r], tmp_v)

            def add_body(k, _, _r=r):
                cur = hist_v[pl.ds(k * 16, 16)]
                hist_v[pl.ds(k * 16, 16)] = cur + tmp_v[pl.ds(k * 16, 16)]
                return _

            lax.fori_loop(0, HBINS // 16, add_body, 0)

        def bank_body(jb, _):
            acc = zeros_i
            for l in range(16):
                acc = acc + hist_v[pl.ds(l * 128 + jb * 16, 16)]
            tmp_v[pl.ds(jb * 16, 16)] = acc
            return _

        lax.fori_loop(0, 8, bank_body, 0)

        lane_lt10 = iota < N_CLS
        pcnt = jnp.zeros((16,), jnp.float32)
        tcnt = jnp.zeros((16,), jnp.float32)
        tcnt_s = []
        for j in range(N_CLS):
            row = plsc.load_gather(tmp_v, [j * N_CLS + iota])
            rowf = jnp.where(lane_lt10, row.astype(jnp.float32), 0.0)
            pcnt = pcnt + rowf
            sj = jnp.sum(rowf)
            tcnt_s.append(sj)
            tcnt = jnp.where(iota == j, sj, tcnt)

        tin = jnp.max(jnp.where(tcnt > 0.0, iota, 0)) + 1

        u_v[...] = pcnt
        sh_idx = jnp.maximum(iota - 1, 0)
        pcnt_sh = plsc.load_gather(u_v, [sh_idx])
        col_live = (iota >= 1) & (iota <= N_CLS)
        cost_v[0, :] = jnp.full((16,), INF)
        for i in range(N_CLS):
            inter_sh = plsc.load_gather(
                tmp_v, [jnp.maximum(i * N_CLS + iota - 1, 0)]
            ).astype(jnp.float32)
            union_sh = tcnt_s[i] + pcnt_sh - inter_sh
            match_sh = inter_sh / jnp.maximum(union_sh, 1.0)
            cost_v[i + 1, :] = jnp.where(col_live, -match_sh, INF)

        u_v[...] = jnp.zeros((16,), jnp.float32)

        def augment(i, carry):
            p, way, v = carry
            p = jnp.where(iota == 0, i, p)
            j0v = zeros_i
            minv = jnp.full((16,), INF)
            used = zeros_i

            def path_cond(st):
                j0v_, p_, _way, _minv, _used, _v = st
                return jnp.sum(jnp.where(iota == j0v_, p_, 0)) != 0

            def path_body(st):
                j0v_, p_, way_, minv_, used_, v_ = st
                used_ = jnp.where(iota == j0v_, 1, used_)
                usedb = used_ != 0
                i0 = jnp.sum(jnp.where(iota == j0v_, p_, 0))
                i0v = jnp.full((16,), i0)
                row = plsc.load_gather(cost_v, [i0v, iota])
                u_all = u_v[...]
                u_i0 = jnp.sum(jnp.where(iota == i0v, u_all, 0.0))
                cur = row - u_i0 - v_
                upd = jnp.logical_and(~usedb, cur < minv_)
                minv_ = jnp.where(upd, cur, minv_)
                way_ = jnp.where(upd, j0v_, way_)
                masked = jnp.where(usedb, INF, minv_)
                delta = jnp.min(masked)
                j1v = plsc.all_reduce_ffs(masked == delta)
                plsc.addupdate_scatter(
                    u_v, [p_], jnp.full((16,), delta), mask=usedb
                )
                v_ = v_ - jnp.where(usedb, delta, 0.0)
                minv_ = jnp.where(usedb, minv_, minv_ - delta)
                return (j1v.astype(jnp.int32), p_, way_, minv_, used_, v_)

            j0v, p, way, minv, used, v = lax.while_loop(
                path_cond, path_body, (j0v, p, way, minv, used, v)
            )

            def unwind_cond(st):
                _p, j0v_ = st
                return jnp.max(j0v_) != 0

            def unwind_body(st):
                p_, j0v_ = st
                j1 = jnp.sum(jnp.where(iota == j0v_, way, 0))
                j1v = jnp.full((16,), j1)
                pj1 = jnp.sum(jnp.where(iota == j1v, p_, 0))
                p_ = jnp.where(iota == j0v_, pj1, p_)
                return (p_, j1v)

            p, _ = lax.while_loop(unwind_cond, unwind_body, (p, j0v))
            return (p, way, v)

        p, _, _ = lax.fori_loop(
            1, N_CLS + 1, augment,
            (zeros_i, zeros_i, jnp.zeros((16,), jnp.float32)),
        )

        ans_v[...] = zeros_i
        valid = (p > 0) & (iota >= 1) & (iota <= N_CLS)
        plsc.store_scatter(
            ans_v, [jnp.maximum(p - 1, 0)], iota - 1, mask=valid
        )
        ans = ans_v[...]
        col = jnp.where((iota < tin) & lane_lt10, ans, 0)
        ans_v[...] = col
        pltpu.sync_copy(ans_v, sh_res.at[b_local, pl.ds(0, 16)])
        ans_v[...] = jnp.full((16,), tin)
        pltpu.sync_copy(ans_v, sh_res.at[b_local, pl.ds(64, 16)])

    plsc.subcore_barrier()

    @pl.when(s == 0)
    def _tin_out():
        acc = zeros_i
        for bl in range(BPC):
            pltpu.sync_copy(sh_res.at[bl, pl.ds(64, 16)], ans_v)
            acc = jnp.where(iota == bl, ans_v[...], acc)
        ans_v[...] = acc
        pltpu.sync_copy(ans_v, tin_hbm.at[c])

    pltpu.sync_copy(sh_res.at[b_local, pl.ds(0, 16)], ans_v)

    pat = []
    for k in range(5):
        pat.append(plsc.load_gather(ans_v, [lax.rem(k * 16 + iota, N_CLS)]))

    def fill_body(g, _):
        base = g * 80
        for j in range(5):
            rep_v[pl.ds(base + j * 16, 16)] = pat[j]
        return _

    lax.fori_loop(0, WORDS_BUF // 80, fill_body, 0)

    for q in range(NDMA):
        pltpu.sync_copy(
            rep_v,
            out_hbm.at[b, pl.ds(chunk * CHUNK * N_CLS + q * WORDS_BUF,
                                WORDS_BUF)],
        )


_miou_sc = pl.kernel(
    _miou_body,
    out_type=_OUT_TYPE,
    mesh=_mesh,
    compiler_params=pltpu.CompilerParams(needs_layout_passes=False),
    scratch_types=_SCRATCH_TYPES,
)


def kernel(predicts, targets):
    out_flat, tin_p = _miou_sc(predicts, targets)
    matching_indices = out_flat.reshape(B, N, N_CLS)
    target_inst_num = jnp.concatenate([tin_p[0, :BPC], tin_p[1, :BPC]])
    return (matching_indices, target_inst_num)


# speedup vs baseline: 5.6102x; 1.4021x over previous
import jax
import jax.numpy as jnp
from jax import lax
from jax.experimental import pallas as pl
from jax.experimental.pallas import tpu as pltpu
from jax.experimental.pallas import tpu_sc as plsc

N_CLS = 10
B = 8
N = 32768
NC = 2
NS = 16
BPC = B // NC
CPB = NS // BPC
CHUNK = N // CPB
HBINS = 16 * 128
INF = float("inf")

_mesh = plsc.VectorSubcoreMesh(
    core_axis_name="c", subcore_axis_name="s", num_cores=NC, num_subcores=NS
)


_OUT_TYPE = [
    jax.ShapeDtypeStruct((B, 16), jnp.int32),
    jax.ShapeDtypeStruct((B, 16), jnp.int32),
]
_SCRATCH_TYPES = [
    pltpu.VMEM((CHUNK,), jnp.int32),
    pltpu.VMEM((CHUNK,), jnp.int32),
    pltpu.VMEM((HBINS,), jnp.int32),
    pltpu.VMEM((HBINS,), jnp.int32),
    pltpu.VMEM((11, 16), jnp.float32),
    pltpu.VMEM((16,), jnp.float32),
    pltpu.VMEM((16,), jnp.int32),
    pltpu.VMEM_SHARED((NS, HBINS), jnp.int32),
]


def _miou_body(predicts_hbm, targets_hbm, col_hbm, tin_hbm,
             t_v, p_v, hist_v, tmp_v, cost_v, u_v, ans_v,
             sh_hist):
    c = lax.axis_index("c")
    s = lax.axis_index("s")
    b_local = s // CPB
    chunk = s % CPB
    b = c * BPC + b_local
    iota = lax.broadcasted_iota(jnp.int32, (16,), 0)
    zeros_i = jnp.zeros((16,), jnp.int32)
    ones_i = jnp.ones((16,), jnp.int32)

    pltpu.sync_copy(targets_hbm.at[b, pl.ds(chunk * CHUNK, CHUNK)], t_v)
    pltpu.sync_copy(predicts_hbm.at[b, pl.ds(chunk * CHUNK, CHUNK)], p_v)

    def zero_body(k, _):
        hist_v[pl.ds(k * 16, 16)] = zeros_i
        return _

    lax.fori_loop(0, HBINS // 16, zero_body, 0)

    bank = iota * 128

    def cnt_body(n, _):
        t = t_v[pl.ds(n * 16, 16)]
        p = p_v[pl.ds(n * 16, 16)]
        idx = bank + t * N_CLS + p
        plsc.addupdate_scatter(hist_v, [idx], ones_i)
        return _

    lax.fori_loop(0, CHUNK // 16, cnt_body, 0)

    pltpu.sync_copy(hist_v, sh_hist.at[s])
    plsc.subcore_barrier()

    @pl.when(chunk == 0)
    def _owner():
        for r in range(1, CPB):
            pltpu.sync_copy(sh_hist.at[s + r], tmp_v)

            def add_body(k, _, _r=r):
                cur = hist_v[pl.ds(k * 16, 16)]
                hist_v[pl.ds(k * 16, 16)] = cur + tmp_v[pl.ds(k * 16, 16)]
                return _

            lax.fori_loop(0, HBINS // 16, add_body, 0)

        def bank_body(jb, _):
            acc = zeros_i
            for l in range(16):
                acc = acc + hist_v[pl.ds(l * 128 + jb * 16, 16)]
            tmp_v[pl.ds(jb * 16, 16)] = acc
            return _

        lax.fori_loop(0, 8, bank_body, 0)

        lane_lt10 = iota < N_CLS
        pcnt = jnp.zeros((16,), jnp.float32)
        tcnt = jnp.zeros((16,), jnp.float32)
        tcnt_s = []
        for j in range(N_CLS):
            row = plsc.load_gather(tmp_v, [j * N_CLS + iota])
            rowf = jnp.where(lane_lt10, row.astype(jnp.float32), 0.0)
            pcnt = pcnt + rowf
            sj = jnp.sum(rowf)
            tcnt_s.append(sj)
            tcnt = jnp.where(iota == j, sj, tcnt)

        tin = jnp.max(jnp.where(tcnt > 0.0, iota, 0)) + 1

        u_v[...] = pcnt
        sh_idx = jnp.maximum(iota - 1, 0)
        pcnt_sh = plsc.load_gather(u_v, [sh_idx])
        col_live = (iota >= 1) & (iota <= N_CLS)
        cost_v[0, :] = jnp.full((16,), INF)
        for i in range(N_CLS):
            inter_sh = plsc.load_gather(
                tmp_v, [jnp.maximum(i * N_CLS + iota - 1, 0)]
            ).astype(jnp.float32)
            union_sh = tcnt_s[i] + pcnt_sh - inter_sh
            match_sh = inter_sh / jnp.maximum(union_sh, 1.0)
            cost_v[i + 1, :] = jnp.where(col_live, -match_sh, INF)

        u_v[...] = jnp.zeros((16,), jnp.float32)

        def augment(i, carry):
            p, way, v = carry
            p = jnp.where(iota == 0, i, p)
            j0v = zeros_i
            minv = jnp.full((16,), INF)
            used = zeros_i

            def path_cond(st):
                j0v_, p_, _way, _minv, _used, _v = st
                return jnp.sum(jnp.where(iota == j0v_, p_, 0)) != 0

            def path_body(st):
                j0v_, p_, way_, minv_, used_, v_ = st
                used_ = jnp.where(iota == j0v_, 1, used_)
                usedb = used_ != 0
                i0 = jnp.sum(jnp.where(iota == j0v_, p_, 0))
                i0v = jnp.full((16,), i0)
                row = plsc.load_gather(cost_v, [i0v, iota])
                u_all = u_v[...]
                u_i0 = jnp.sum(jnp.where(iota == i0v, u_all, 0.0))
                cur = row - u_i0 - v_
                upd = jnp.logical_and(~usedb, cur < minv_)
                minv_ = jnp.where(upd, cur, minv_)
                way_ = jnp.where(upd, j0v_, way_)
                masked = jnp.where(usedb, INF, minv_)
                delta = jnp.min(masked)
                j1v = plsc.all_reduce_ffs(masked == delta)
                plsc.addupdate_scatter(
                    u_v, [p_], jnp.full((16,), delta), mask=usedb
                )
                v_ = v_ - jnp.where(usedb, delta, 0.0)
                minv_ = jnp.where(usedb, minv_, minv_ - delta)
                return (j1v.astype(jnp.int32), p_, way_, minv_, used_, v_)

            j0v, p, way, minv, used, v = lax.while_loop(
                path_cond, path_body, (j0v, p, way, minv, used, v)
            )

            def unwind_cond(st):
                _p, j0v_ = st
                return jnp.max(j0v_) != 0

            def unwind_body(st):
                p_, j0v_ = st
                j1 = jnp.sum(jnp.where(iota == j0v_, way, 0))
                j1v = jnp.full((16,), j1)
                pj1 = jnp.sum(jnp.where(iota == j1v, p_, 0))
                p_ = jnp.where(iota == j0v_, pj1, p_)
                return (p_, j1v)

            p, _ = lax.while_loop(unwind_cond, unwind_body, (p, j0v))
            return (p, way, v)

        p, _, _ = lax.fori_loop(
            1, N_CLS + 1, augment,
            (zeros_i, zeros_i, jnp.zeros((16,), jnp.float32)),
        )

        ans_v[...] = zeros_i
        valid = (p > 0) & (iota >= 1) & (iota <= N_CLS)
        plsc.store_scatter(
            ans_v, [jnp.maximum(p - 1, 0)], iota - 1, mask=valid
        )
        ans = ans_v[...]
        col = jnp.where((iota < tin) & lane_lt10, ans, 0)
        ans_v[...] = col
        pltpu.sync_copy(ans_v, col_hbm.at[b])
        ans_v[...] = jnp.full((16,), tin)
        pltpu.sync_copy(ans_v, tin_hbm.at[b])


_miou_sc = pl.kernel(
    _miou_body,
    out_type=_OUT_TYPE,
    mesh=_mesh,
    compiler_params=pltpu.CompilerParams(needs_layout_passes=False),
    scratch_types=_SCRATCH_TYPES,
)


BLK = 4096


def _bcast_body(col_ref, out_ref):
    b = pl.program_id(0)
    row = col_ref[pl.ds(b, 1), 0:N_CLS]
    out_ref[...] = jnp.broadcast_to(row[:, None, :], (1, BLK, N_CLS))


_bcast_tc = pl.pallas_call(
    _bcast_body,
    grid=(B, N // BLK),
    in_specs=[pl.BlockSpec((B, 16), lambda b, i: (0, 0))],
    out_specs=pl.BlockSpec((1, BLK, N_CLS), lambda b, i: (b, i, 0)),
    out_shape=jax.ShapeDtypeStruct((B, N, N_CLS), jnp.int32),
)


def kernel(predicts, targets):
    col8, tin8 = _miou_sc(predicts, targets)
    matching_indices = _bcast_tc(col8)
    target_inst_num = tin8[:, 0]
    return (matching_indices, target_inst_num)


# speedup vs baseline: 21.1082x; 3.7625x over previous
import jax
import jax.numpy as jnp
from jax import lax
from jax.experimental import pallas as pl
from jax.experimental.pallas import tpu as pltpu
from jax.experimental.pallas import tpu_sc as plsc

N_CLS = 10
B = 8
N = 32768
NC = 2
NS = 16
BPC = B // NC
CPB = NS // BPC
CHUNK = N // CPB
HBINS = 16 * 128
INF = float("inf")

_mesh = plsc.VectorSubcoreMesh(
    core_axis_name="c", subcore_axis_name="s", num_cores=NC, num_subcores=NS
)


_OUT_TYPE = [
    jax.ShapeDtypeStruct((B, 16), jnp.int32),
    jax.ShapeDtypeStruct((B, 16), jnp.int32),
]
_SCRATCH_TYPES = [
    pltpu.VMEM((CHUNK,), jnp.int32),
    pltpu.VMEM((CHUNK,), jnp.int32),
    pltpu.VMEM((HBINS,), jnp.int32),
    pltpu.VMEM((HBINS,), jnp.int32),
    pltpu.VMEM((11, 16), jnp.float32),
    pltpu.VMEM((16,), jnp.float32),
    pltpu.VMEM((16,), jnp.int32),
    pltpu.VMEM_SHARED((NS, HBINS), jnp.int32),
]


def _miou_body(predicts_hbm, targets_hbm, col_hbm, tin_hbm,
             t_v, p_v, hist_v, tmp_v, cost_v, u_v, ans_v,
             sh_hist):
    c = lax.axis_index("c")
    s = lax.axis_index("s")
    b_local = s // CPB
    chunk = s % CPB
    b = c * BPC + b_local
    iota = lax.broadcasted_iota(jnp.int32, (16,), 0)
    zeros_i = jnp.zeros((16,), jnp.int32)
    ones_i = jnp.ones((16,), jnp.int32)

    pltpu.sync_copy(targets_hbm.at[b, pl.ds(chunk * CHUNK, CHUNK)], t_v)
    pltpu.sync_copy(predicts_hbm.at[b, pl.ds(chunk * CHUNK, CHUNK)], p_v)

    def zero_body(k, _):
        hist_v[pl.ds(k * 16, 16)] = zeros_i
        return _

    lax.fori_loop(0, HBINS // 16, zero_body, 0)

    bank = iota * 128

    def cnt_body(n, _):
        t = t_v[pl.ds(n * 16, 16)]
        p = p_v[pl.ds(n * 16, 16)]
        idx = bank + t * N_CLS + p
        plsc.addupdate_scatter(hist_v, [idx], ones_i)
        return _

    lax.fori_loop(0, CHUNK // 16, cnt_body, 0)

    pltpu.sync_copy(hist_v, sh_hist.at[s])
    plsc.subcore_barrier()

    @pl.when(chunk == 0)
    def _owner():
        for r in range(1, CPB):
            pltpu.sync_copy(sh_hist.at[s + r], tmp_v)

            def add_body(k, _, _r=r):
                cur = hist_v[pl.ds(k * 16, 16)]
                hist_v[pl.ds(k * 16, 16)] = cur + tmp_v[pl.ds(k * 16, 16)]
                return _

            lax.fori_loop(0, HBINS // 16, add_body, 0)

        def bank_body(jb, _):
            acc = zeros_i
            for l in range(16):
                acc = acc + hist_v[pl.ds(l * 128 + jb * 16, 16)]
            tmp_v[pl.ds(jb * 16, 16)] = acc
            return _

        lax.fori_loop(0, 8, bank_body, 0)

        lane_lt10 = iota < N_CLS
        pcnt = jnp.zeros((16,), jnp.float32)
        tcnt = jnp.zeros((16,), jnp.float32)
        tcnt_s = []
        for j in range(N_CLS):
            row = plsc.load_gather(tmp_v, [j * N_CLS + iota])
            rowf = jnp.where(lane_lt10, row.astype(jnp.float32), 0.0)
            pcnt = pcnt + rowf
            sj = jnp.sum(rowf)
            tcnt_s.append(sj)
            tcnt = jnp.where(iota == j, sj, tcnt)

        tin = jnp.max(jnp.where(tcnt > 0.0, iota, 0)) + 1

        u_v[...] = pcnt
        sh_idx = jnp.maximum(iota - 1, 0)
        pcnt_sh = plsc.load_gather(u_v, [sh_idx])
        col_live = (iota >= 1) & (iota <= N_CLS)
        cost_v[0, :] = jnp.full((16,), INF)
        for i in range(N_CLS):
            inter_sh = plsc.load_gather(
                tmp_v, [jnp.maximum(i * N_CLS + iota - 1, 0)]
            ).astype(jnp.float32)
            union_sh = tcnt_s[i] + pcnt_sh - inter_sh
            match_sh = inter_sh / jnp.maximum(union_sh, 1.0)
            cost_v[i + 1, :] = jnp.where(col_live, -match_sh, INF)

        u_v[...] = jnp.zeros((16,), jnp.float32)

        def augment(i, carry):
            p, way, v = carry
            p = jnp.where(iota == 0, i, p)
            j0v = zeros_i
            minv = jnp.full((16,), INF)
            used = zeros_i

            def path_cond(st):
                j0v_, p_, _way, _minv, _used, _v = st
                return jnp.sum(jnp.where(iota == j0v_, p_, 0)) != 0

            def path_body(st):
                j0v_, p_, way_, minv_, used_, v_ = st
                used_ = jnp.where(iota == j0v_, 1, used_)
                usedb = used_ != 0
                i0 = jnp.sum(jnp.where(iota == j0v_, p_, 0))
                i0v = jnp.full((16,), i0)
                row = plsc.load_gather(cost_v, [i0v, iota])
                u_all = u_v[...]
                u_i0 = jnp.sum(jnp.where(iota == i0v, u_all, 0.0))
                cur = row - u_i0 - v_
                upd = jnp.logical_and(~usedb, cur < minv_)
                minv_ = jnp.where(upd, cur, minv_)
                way_ = jnp.where(upd, j0v_, way_)
                masked = jnp.where(usedb, INF, minv_)
                delta = jnp.min(masked)
                j1v = plsc.all_reduce_ffs(masked == delta)
                plsc.addupdate_scatter(
                    u_v, [p_], jnp.full((16,), delta), mask=usedb
                )
                v_ = v_ - jnp.where(usedb, delta, 0.0)
                minv_ = jnp.where(usedb, minv_, minv_ - delta)
                return (j1v.astype(jnp.int32), p_, way_, minv_, used_, v_)

            j0v, p, way, minv, used, v = lax.while_loop(
                path_cond, path_body, (j0v, p, way, minv, used, v)
            )

            def unwind_cond(st):
                _p, j0v_ = st
                return jnp.max(j0v_) != 0

            def unwind_body(st):
                p_, j0v_ = st
                j1 = jnp.sum(jnp.where(iota == j0v_, way, 0))
                j1v = jnp.full((16,), j1)
                pj1 = jnp.sum(jnp.where(iota == j1v, p_, 0))
                p_ = jnp.where(iota == j0v_, pj1, p_)
                return (p_, j1v)

            p, _ = lax.while_loop(unwind_cond, unwind_body, (p, j0v))
            return (p, way, v)

        p, _, _ = lax.fori_loop(
            1, N_CLS + 1, augment,
            (zeros_i, zeros_i, jnp.zeros((16,), jnp.float32)),
        )

        ans_v[...] = zeros_i
        valid = (p > 0) & (iota >= 1) & (iota <= N_CLS)
        plsc.store_scatter(
            ans_v, [jnp.maximum(p - 1, 0)], iota - 1, mask=valid
        )
        ans = ans_v[...]
        col = jnp.where((iota < tin) & lane_lt10, ans, 0)
        ans_v[...] = col
        pltpu.sync_copy(ans_v, col_hbm.at[b])
        ans_v[...] = jnp.full((16,), tin)
        pltpu.sync_copy(ans_v, tin_hbm.at[b])


_miou_sc = pl.kernel(
    _miou_body,
    out_type=_OUT_TYPE,
    mesh=_mesh,
    compiler_params=pltpu.CompilerParams(needs_layout_passes=False),
    scratch_types=_SCRATCH_TYPES,
)


BLK = 4096


def _bcast_body(col_ref, out_ref):
    colt = col_ref[...].T[0:N_CLS, :]
    out_ref[...] = jnp.broadcast_to(colt[:, :, None], (N_CLS, B, BLK))


_bcast_tc = pl.pallas_call(
    _bcast_body,
    grid=(N // BLK,),
    in_specs=[pl.BlockSpec((B, 16), lambda i: (0, 0))],
    out_specs=pl.BlockSpec((N_CLS, B, BLK), lambda i: (0, 0, i)),
    out_shape=jax.ShapeDtypeStruct((N_CLS, B, N), jnp.int32),
)


def kernel(predicts, targets):
    col8, tin8 = _miou_sc(predicts, targets)
    matching_indices = jnp.transpose(_bcast_tc(col8), (1, 2, 0))
    target_inst_num = tin8[:, 0]
    return (matching_indices, target_inst_num)


# speedup vs baseline: 24.6662x; 1.1686x over previous
import jax
import jax.numpy as jnp
from jax import lax
from jax.experimental import pallas as pl
from jax.experimental.pallas import tpu as pltpu
from jax.experimental.pallas import tpu_sc as plsc

N_CLS = 10
B = 8
N = 32768
NC = 2
NS = 16
BPC = B // NC
CPB = NS // BPC
CHUNK = N // CPB
HBINS = 16 * 128
INF = float("inf")

_mesh = plsc.VectorSubcoreMesh(
    core_axis_name="c", subcore_axis_name="s", num_cores=NC, num_subcores=NS
)


_OUT_TYPE = [
    jax.ShapeDtypeStruct((B, 16), jnp.int32),
    jax.ShapeDtypeStruct((B, 16), jnp.int32),
]
_SCRATCH_TYPES = [
    pltpu.VMEM((CHUNK,), jnp.int32),
    pltpu.VMEM((CHUNK,), jnp.int32),
    pltpu.VMEM((HBINS,), jnp.int32),
    pltpu.VMEM((HBINS,), jnp.int32),
    pltpu.VMEM((11, 16), jnp.float32),
    pltpu.VMEM((16,), jnp.float32),
    pltpu.VMEM((16,), jnp.int32),
    pltpu.VMEM_SHARED((NS, HBINS), jnp.int32),
    pltpu.SemaphoreType.DMA,
    pltpu.SemaphoreType.DMA,
]


def _miou_body(predicts_hbm, targets_hbm, col_hbm, tin_hbm,
             t_v, p_v, hist_v, tmp_v, cost_v, u_v, ans_v,
             sh_hist, sem_t, sem_p):
    c = lax.axis_index("c")
    s = lax.axis_index("s")
    b_local = s // CPB
    chunk = s % CPB
    b = c * BPC + b_local
    iota = lax.broadcasted_iota(jnp.int32, (16,), 0)
    zeros_i = jnp.zeros((16,), jnp.int32)
    ones_i = jnp.ones((16,), jnp.int32)

    cp_t = pltpu.async_copy(targets_hbm.at[b, pl.ds(chunk * CHUNK, CHUNK)],
                            t_v, sem_t)
    cp_p = pltpu.async_copy(predicts_hbm.at[b, pl.ds(chunk * CHUNK, CHUNK)],
                            p_v, sem_p)

    @plsc.parallel_loop(0, HBINS // 16, unroll=8)
    def _zero(k):
        hist_v[pl.ds(k * 16, 16)] = zeros_i

    cp_t.wait()
    cp_p.wait()

    bank = iota * 128

    @plsc.parallel_loop(0, CHUNK // 16, unroll=8)
    def _count(n):
        t = t_v[pl.ds(n * 16, 16)]
        p = p_v[pl.ds(n * 16, 16)]
        idx = bank + t * N_CLS + p
        plsc.addupdate_scatter(hist_v, [idx], ones_i)

    pltpu.sync_copy(hist_v, sh_hist.at[s])
    plsc.subcore_barrier()

    @pl.when(chunk == 0)
    def _owner():
        for r in range(1, CPB):
            pltpu.sync_copy(sh_hist.at[s + r], tmp_v)

            @plsc.parallel_loop(0, HBINS // 16, unroll=8)
            def _add(k, _r=r):
                cur = hist_v[pl.ds(k * 16, 16)]
                hist_v[pl.ds(k * 16, 16)] = cur + tmp_v[pl.ds(k * 16, 16)]

        @plsc.parallel_loop(0, 8, unroll=2)
        def _bank(jb):
            acc = zeros_i
            for l in range(16):
                acc = acc + hist_v[pl.ds(l * 128 + jb * 16, 16)]
            tmp_v[pl.ds(jb * 16, 16)] = acc

        lane_lt10 = iota < N_CLS
        pcnt = jnp.zeros((16,), jnp.float32)
        tcnt = jnp.zeros((16,), jnp.float32)
        tcnt_s = []
        for j in range(N_CLS):
            row = plsc.load_gather(tmp_v, [j * N_CLS + iota])
            rowf = jnp.where(lane_lt10, row.astype(jnp.float32), 0.0)
            pcnt = pcnt + rowf
            sj = jnp.sum(rowf)
            tcnt_s.append(sj)
            tcnt = jnp.where(iota == j, sj, tcnt)

        tin = jnp.max(jnp.where(tcnt > 0.0, iota, 0)) + 1

        u_v[...] = pcnt
        sh_idx = jnp.maximum(iota - 1, 0)
        pcnt_sh = plsc.load_gather(u_v, [sh_idx])
        col_live = (iota >= 1) & (iota <= N_CLS)
        cost_v[0, :] = jnp.full((16,), INF)
        for i in range(N_CLS):
            inter_sh = plsc.load_gather(
                tmp_v, [jnp.maximum(i * N_CLS + iota - 1, 0)]
            ).astype(jnp.float32)
            union_sh = tcnt_s[i] + pcnt_sh - inter_sh
            match_sh = inter_sh / jnp.maximum(union_sh, 1.0)
            cost_v[i + 1, :] = jnp.where(col_live, -match_sh, INF)

        u_v[...] = jnp.zeros((16,), jnp.float32)

        def augment(i, carry):
            p, way, v = carry
            p = jnp.where(iota == 0, i, p)
            j0v = zeros_i
            minv = jnp.full((16,), INF)
            used = zeros_i

            def path_cond(st):
                j0v_, p_, _way, _minv, _used, _v = st
                return jnp.sum(jnp.where(iota == j0v_, p_, 0)) != 0

            def path_body(st):
                j0v_, p_, way_, minv_, used_, v_ = st
                used_ = jnp.where(iota == j0v_, 1, used_)
                usedb = used_ != 0
                i0 = jnp.sum(jnp.where(iota == j0v_, p_, 0))
                i0v = jnp.full((16,), i0)
                row = plsc.load_gather(cost_v, [i0v, iota])
                u_all = u_v[...]
                u_i0 = jnp.sum(jnp.where(iota == i0v, u_all, 0.0))
                cur = row - u_i0 - v_
                upd = jnp.logical_and(~usedb, cur < minv_)
                minv_ = jnp.where(upd, cur, minv_)
                way_ = jnp.where(upd, j0v_, way_)
                masked = jnp.where(usedb, INF, minv_)
                delta = jnp.min(masked)
                j1v = plsc.all_reduce_ffs(masked == delta)
                plsc.addupdate_scatter(
                    u_v, [p_], jnp.full((16,), delta), mask=usedb
                )
                v_ = v_ - jnp.where(usedb, delta, 0.0)
                minv_ = jnp.where(usedb, minv_, minv_ - delta)
                return (j1v.astype(jnp.int32), p_, way_, minv_, used_, v_)

            j0v, p, way, minv, used, v = lax.while_loop(
                path_cond, path_body, (j0v, p, way, minv, used, v)
            )

            def unwind_cond(st):
                _p, j0v_ = st
                return jnp.max(j0v_) != 0

            def unwind_body(st):
                p_, j0v_ = st
                j1 = jnp.sum(jnp.where(iota == j0v_, way, 0))
                j1v = jnp.full((16,), j1)
                pj1 = jnp.sum(jnp.where(iota == j1v, p_, 0))
                p_ = jnp.where(iota == j0v_, pj1, p_)
                return (p_, j1v)

            p, _ = lax.while_loop(unwind_cond, unwind_body, (p, j0v))
            return (p, way, v)

        p, _, _ = lax.fori_loop(
            1, N_CLS + 1, augment,
            (zeros_i, zeros_i, jnp.zeros((16,), jnp.float32)),
        )

        ans_v[...] = zeros_i
        valid = (p > 0) & (iota >= 1) & (iota <= N_CLS)
        plsc.store_scatter(
            ans_v, [jnp.maximum(p - 1, 0)], iota - 1, mask=valid
        )
        ans = ans_v[...]
        col = jnp.where((iota < tin) & lane_lt10, ans, 0)
        ans_v[...] = col
        pltpu.sync_copy(ans_v, col_hbm.at[b])
        ans_v[...] = jnp.full((16,), tin)
        pltpu.sync_copy(ans_v, tin_hbm.at[b])


_miou_sc = pl.kernel(
    _miou_body,
    out_type=_OUT_TYPE,
    mesh=_mesh,
    compiler_params=pltpu.CompilerParams(needs_layout_passes=False),
    scratch_types=_SCRATCH_TYPES,
)


BLK = 4096


def _bcast_body(col_ref, out_ref):
    colt = col_ref[...].T[0:N_CLS, :]
    out_ref[...] = jnp.broadcast_to(colt[:, :, None], (N_CLS, B, BLK))


_bcast_tc = pl.pallas_call(
    _bcast_body,
    grid=(N // BLK,),
    in_specs=[pl.BlockSpec((B, 16), lambda i: (0, 0))],
    out_specs=pl.BlockSpec((N_CLS, B, BLK), lambda i: (0, 0, i)),
    out_shape=jax.ShapeDtypeStruct((N_CLS, B, N), jnp.int32),
)


def kernel(predicts, targets):
    col8, tin8 = _miou_sc(predicts, targets)
    matching_indices = jnp.transpose(_bcast_tc(col8), (1, 2, 0))
    target_inst_num = tin8[:, 0]
    return (matching_indices, target_inst_num)


# speedup vs baseline: 26.8253x; 1.0875x over previous
import jax
import jax.numpy as jnp
from jax import lax
from jax.experimental import pallas as pl
from jax.experimental.pallas import tpu as pltpu
from jax.experimental.pallas import tpu_sc as plsc

N_CLS = 10
B = 8
N = 32768
NC = 2
NS = 16
BPC = B // NC
CPB = NS // BPC
CHUNK = N // CPB
HBINS = 16 * 128
INF = float("inf")

_mesh = plsc.VectorSubcoreMesh(
    core_axis_name="c", subcore_axis_name="s", num_cores=NC, num_subcores=NS
)


_OUT_TYPE = [
    jax.ShapeDtypeStruct((B, 16), jnp.int32),
    jax.ShapeDtypeStruct((B, 16), jnp.int32),
]
_SCRATCH_TYPES = [
    pltpu.VMEM((CHUNK,), jnp.int32),
    pltpu.VMEM((CHUNK,), jnp.int32),
    pltpu.VMEM((HBINS,), jnp.int32),
    pltpu.VMEM((HBINS,), jnp.int32),
    pltpu.VMEM((11, 16), jnp.float32),
    pltpu.VMEM((16,), jnp.float32),
    pltpu.VMEM((16,), jnp.int32),
    pltpu.VMEM_SHARED((NS, HBINS), jnp.int32),
    pltpu.SemaphoreType.DMA,
    pltpu.SemaphoreType.DMA,
]


def _miou_body(predicts_hbm, targets_hbm, col_hbm, tin_hbm,
             t_v, p_v, hist_v, tmp_v, cost_v, u_v, ans_v,
             sh_hist, sem_t, sem_p):
    c = lax.axis_index("c")
    s = lax.axis_index("s")
    b_local = s // CPB
    chunk = s % CPB
    b = c * BPC + b_local
    iota = lax.broadcasted_iota(jnp.int32, (16,), 0)
    zeros_i = jnp.zeros((16,), jnp.int32)
    ones_i = jnp.ones((16,), jnp.int32)

    cp_t = pltpu.async_copy(targets_hbm.at[b, pl.ds(chunk * CHUNK, CHUNK)],
                            t_v, sem_t)
    cp_p = pltpu.async_copy(predicts_hbm.at[b, pl.ds(chunk * CHUNK, CHUNK)],
                            p_v, sem_p)

    @plsc.parallel_loop(0, HBINS // 16, unroll=8)
    def _zero(k):
        hist_v[pl.ds(k * 16, 16)] = zeros_i

    cp_t.wait()
    cp_p.wait()

    bank = iota * 128

    @plsc.parallel_loop(0, CHUNK // 16, unroll=8)
    def _count(n):
        t = t_v[pl.ds(n * 16, 16)]
        p = p_v[pl.ds(n * 16, 16)]
        idx = bank + t * N_CLS + p
        plsc.addupdate_scatter(hist_v, [idx], ones_i)

    pltpu.sync_copy(hist_v, sh_hist.at[s])
    plsc.subcore_barrier()

    @pl.when(chunk == 0)
    def _owner():
        for r in range(1, CPB):
            pltpu.sync_copy(sh_hist.at[s + r], tmp_v)

            @plsc.parallel_loop(0, HBINS // 16, unroll=8)
            def _add(k, _r=r):
                cur = hist_v[pl.ds(k * 16, 16)]
                hist_v[pl.ds(k * 16, 16)] = cur + tmp_v[pl.ds(k * 16, 16)]

        @plsc.parallel_loop(0, 8, unroll=2)
        def _bank(jb):
            acc = zeros_i
            for l in range(16):
                acc = acc + hist_v[pl.ds(l * 128 + jb * 16, 16)]
            tmp_v[pl.ds(jb * 16, 16)] = acc

        lane_lt10 = iota < N_CLS
        pcnt = jnp.zeros((16,), jnp.float32)
        tcnt = jnp.zeros((16,), jnp.float32)
        tcnt_s = []
        for j in range(N_CLS):
            row = plsc.load_gather(tmp_v, [j * N_CLS + iota])
            rowf = jnp.where(lane_lt10, row.astype(jnp.float32), 0.0)
            pcnt = pcnt + rowf
            sj = jnp.sum(rowf)
            tcnt_s.append(sj)
            tcnt = jnp.where(iota == j, sj, tcnt)

        tin = jnp.max(jnp.where(tcnt > 0.0, iota, 0)) + 1

        u_v[...] = pcnt
        sh_idx = jnp.maximum(iota - 1, 0)
        pcnt_sh = plsc.load_gather(u_v, [sh_idx])
        col_live = (iota >= 1) & (iota <= N_CLS)
        cost_v[0, :] = jnp.full((16,), INF)
        for i in range(N_CLS):
            inter_sh = plsc.load_gather(
                tmp_v, [jnp.maximum(i * N_CLS + iota - 1, 0)]
            ).astype(jnp.float32)
            union_sh = tcnt_s[i] + pcnt_sh - inter_sh
            match_sh = inter_sh / jnp.maximum(union_sh, 1.0)
            cost_v[i + 1, :] = jnp.where(col_live, -match_sh, INF)

        u_v[...] = jnp.zeros((16,), jnp.float32)

        def augment(i, carry):
            p, way, v = carry
            p = jnp.where(iota == 0, i, p)
            j0v = zeros_i
            minv = jnp.full((16,), INF)
            used = zeros_i

            def path_cond(st):
                j0v_, p_, _way, _minv, _used, _v = st
                return jnp.sum(jnp.where(iota == j0v_, p_, 0)) != 0

            def path_body(st):
                j0v_, p_, way_, minv_, used_, v_ = st
                used_ = jnp.where(iota == j0v_, 1, used_)
                usedb = used_ != 0
                i0 = jnp.sum(jnp.where(iota == j0v_, p_, 0))
                i0v = jnp.full((16,), i0)
                row = plsc.load_gather(cost_v, [i0v, iota])
                u_all = u_v[...]
                u_i0 = jnp.sum(jnp.where(iota == i0v, u_all, 0.0))
                cur = row - u_i0 - v_
                upd = jnp.logical_and(~usedb, cur < minv_)
                minv_ = jnp.where(upd, cur, minv_)
                way_ = jnp.where(upd, j0v_, way_)
                masked = jnp.where(usedb, INF, minv_)
                delta = jnp.min(masked)
                j1v = plsc.all_reduce_ffs(masked == delta)
                plsc.addupdate_scatter(
                    u_v, [p_], jnp.full((16,), delta), mask=usedb
                )
                v_ = v_ - jnp.where(usedb, delta, 0.0)
                minv_ = jnp.where(usedb, minv_, minv_ - delta)
                return (j1v.astype(jnp.int32), p_, way_, minv_, used_, v_)

            j0v, p, way, minv, used, v = lax.while_loop(
                path_cond, path_body, (j0v, p, way, minv, used, v)
            )

            def unwind_cond(st):
                _p, j0v_ = st
                return jnp.max(j0v_) != 0

            def unwind_body(st):
                p_, j0v_ = st
                j1 = jnp.sum(jnp.where(iota == j0v_, way, 0))
                j1v = jnp.full((16,), j1)
                pj1 = jnp.sum(jnp.where(iota == j1v, p_, 0))
                p_ = jnp.where(iota == j0v_, pj1, p_)
                return (p_, j1v)

            p, _ = lax.while_loop(unwind_cond, unwind_body, (p, j0v))
            return (p, way, v)

        p, _, _ = lax.fori_loop(
            1, N_CLS + 1, augment,
            (zeros_i, zeros_i, jnp.zeros((16,), jnp.float32)),
        )

        ans_v[...] = zeros_i
        valid = (p > 0) & (iota >= 1) & (iota <= N_CLS)
        plsc.store_scatter(
            ans_v, [jnp.maximum(p - 1, 0)], iota - 1, mask=valid
        )
        ans = ans_v[...]
        col = jnp.where((iota < tin) & lane_lt10, ans, 0)
        ans_v[...] = col
        pltpu.sync_copy(ans_v, col_hbm.at[b])
        ans_v[...] = jnp.full((16,), tin)
        pltpu.sync_copy(ans_v, tin_hbm.at[b])


_miou_sc = pl.kernel(
    _miou_body,
    out_type=_OUT_TYPE,
    mesh=_mesh,
    compiler_params=pltpu.CompilerParams(needs_layout_passes=False),
    scratch_types=_SCRATCH_TYPES,
)


BLK = 8192


def _bcast_body(col_ref, tin_ref, out_ref, tin_out_ref):
    colt = col_ref[...].T[0:N_CLS, :]
    out_ref[...] = jnp.broadcast_to(colt[:, :, None], (N_CLS, B, BLK))

    @pl.when(pl.program_id(0) == 0)
    def _tin():
        tin_out_ref[...] = tin_ref[...].T[0, :]


_bcast_tc = pl.pallas_call(
    _bcast_body,
    grid=(N // BLK,),
    in_specs=[
        pl.BlockSpec((B, 16), lambda i: (0, 0)),
        pl.BlockSpec((B, 16), lambda i: (0, 0)),
    ],
    out_specs=[
        pl.BlockSpec((N_CLS, B, BLK), lambda i: (0, 0, i)),
        pl.BlockSpec((B,), lambda i: (0,)),
    ],
    out_shape=[
        jax.ShapeDtypeStruct((N_CLS, B, N), jnp.int32),
        jax.ShapeDtypeStruct((B,), jnp.int32),
    ],
)


def kernel(predicts, targets):
    col8, tin8 = _miou_sc(predicts, targets)
    planes, target_inst_num = _bcast_tc(col8, tin8)
    matching_indices = jnp.transpose(planes, (1, 2, 0))
    return (matching_indices, target_inst_num)


# speedup vs baseline: 26.8273x; 1.0001x over previous
import jax
import jax.numpy as jnp
from jax import lax
from jax.experimental import pallas as pl
from jax.experimental.pallas import tpu as pltpu
from jax.experimental.pallas import tpu_sc as plsc

N_CLS = 10
B = 8
N = 32768
NC = 2
NS = 16
BPC = B // NC
CPB = NS // BPC
CHUNK = N // CPB
HBINS = 16 * 128
INF = float("inf")

_mesh = plsc.VectorSubcoreMesh(
    core_axis_name="c", subcore_axis_name="s", num_cores=NC, num_subcores=NS
)


_OUT_TYPE = [
    jax.ShapeDtypeStruct((B, 16), jnp.int32),
    jax.ShapeDtypeStruct((B, 16), jnp.int32),
]
_SCRATCH_TYPES = [
    pltpu.VMEM((CHUNK,), jnp.int32),
    pltpu.VMEM((CHUNK,), jnp.int32),
    pltpu.VMEM((HBINS,), jnp.int32),
    pltpu.VMEM((HBINS,), jnp.int32),
    pltpu.VMEM((11, 16), jnp.float32),
    pltpu.VMEM((16,), jnp.float32),
    pltpu.VMEM((16,), jnp.int32),
    pltpu.VMEM_SHARED((NS, HBINS), jnp.int32),
    pltpu.SemaphoreType.DMA,
    pltpu.SemaphoreType.DMA,
]


def _miou_body(predicts_hbm, targets_hbm, col_hbm, tin_hbm,
             t_v, p_v, hist_v, tmp_v, cost_v, u_v, ans_v,
             sh_hist, sem_t, sem_p):
    c = lax.axis_index("c")
    s = lax.axis_index("s")
    b_local = s // CPB
    chunk = s % CPB
    b = c * BPC + b_local
    iota = lax.broadcasted_iota(jnp.int32, (16,), 0)
    zeros_i = jnp.zeros((16,), jnp.int32)
    ones_i = jnp.ones((16,), jnp.int32)

    cp_t = pltpu.async_copy(targets_hbm.at[b, pl.ds(chunk * CHUNK, CHUNK)],
                            t_v, sem_t)
    cp_p = pltpu.async_copy(predicts_hbm.at[b, pl.ds(chunk * CHUNK, CHUNK)],
                            p_v, sem_p)

    @plsc.parallel_loop(0, HBINS // 16, unroll=8)
    def _zero(k):
        hist_v[pl.ds(k * 16, 16)] = zeros_i

    cp_t.wait()
    cp_p.wait()

    bank = iota * 128

    @plsc.parallel_loop(0, CHUNK // 16, unroll=8)
    def _count(n):
        t = t_v[pl.ds(n * 16, 16)]
        p = p_v[pl.ds(n * 16, 16)]
        idx = bank + t * N_CLS + p
        plsc.addupdate_scatter(hist_v, [idx], ones_i)

    pltpu.sync_copy(hist_v, sh_hist.at[s])
    plsc.subcore_barrier()

    @pl.when(chunk == 0)
    def _owner():
        for r in range(1, CPB):
            pltpu.sync_copy(sh_hist.at[s + r], tmp_v)

            @plsc.parallel_loop(0, HBINS // 16, unroll=8)
            def _add(k, _r=r):
                cur = hist_v[pl.ds(k * 16, 16)]
                hist_v[pl.ds(k * 16, 16)] = cur + tmp_v[pl.ds(k * 16, 16)]

        @plsc.parallel_loop(0, 8, unroll=2)
        def _bank(jb):
            acc = zeros_i
            for l in range(16):
                acc = acc + hist_v[pl.ds(l * 128 + jb * 16, 16)]
            tmp_v[pl.ds(jb * 16, 16)] = acc

        lane_lt10 = iota < N_CLS
        pcnt = jnp.zeros((16,), jnp.float32)
        tcnt = jnp.zeros((16,), jnp.float32)
        tcnt_s = []
        for j in range(N_CLS):
            row = plsc.load_gather(tmp_v, [j * N_CLS + iota])
            rowf = jnp.where(lane_lt10, row.astype(jnp.float32), 0.0)
            pcnt = pcnt + rowf
            sj = jnp.sum(rowf)
            tcnt_s.append(sj)
            tcnt = jnp.where(iota == j, sj, tcnt)

        tin = jnp.max(jnp.where(tcnt > 0.0, iota, 0)) + 1

        u_v[...] = pcnt
        sh_idx = jnp.maximum(iota - 1, 0)
        pcnt_sh = plsc.load_gather(u_v, [sh_idx])
        col_live = (iota >= 1) & (iota <= N_CLS)
        cost_v[0, :] = jnp.full((16,), INF)
        for i in range(N_CLS):
            inter_sh = plsc.load_gather(
                tmp_v, [jnp.maximum(i * N_CLS + iota - 1, 0)]
            ).astype(jnp.float32)
            union_sh = tcnt_s[i] + pcnt_sh - inter_sh
            match_sh = inter_sh / jnp.maximum(union_sh, 1.0)
            cost_v[i + 1, :] = jnp.where(col_live, -match_sh, INF)

        u_v[...] = jnp.zeros((16,), jnp.float32)

        def augment(i, carry):
            p, way, v = carry
            p = jnp.where(iota == 0, i, p)
            ans_v[...] = p
            j0v = zeros_i
            pj0v = jnp.full((16,), i)
            minv = jnp.full((16,), INF)
            used = zeros_i

            def path_cond(st):
                _j0v, pj0v_, _way, _minv, _used, _v = st
                return jnp.max(pj0v_) != 0

            def path_body(st):
                j0v_, pj0v_, way_, minv_, used_, v_ = st
                used_ = jnp.where(iota == j0v_, 1, used_)
                usedb = used_ != 0
                i0v = pj0v_
                row = plsc.load_gather(cost_v, [i0v, iota])
                u_i0v = plsc.load_gather(u_v, [i0v])
                cur = row - u_i0v - v_
                upd = jnp.logical_and(~usedb, cur < minv_)
                minv_ = jnp.where(upd, cur, minv_)
                way_ = jnp.where(upd, j0v_, way_)
                masked = jnp.where(usedb, INF, minv_)
                delta = jnp.min(masked)
                j1v = plsc.all_reduce_ffs(masked == delta).astype(jnp.int32)
                plsc.addupdate_scatter(
                    u_v, [p], jnp.full((16,), delta), mask=usedb
                )
                v_ = v_ - jnp.where(usedb, delta, 0.0)
                minv_ = jnp.where(usedb, minv_, minv_ - delta)
                pj1v = plsc.load_gather(ans_v, [j1v])
                return (j1v, pj1v, way_, minv_, used_, v_)

            j0v, _, way, minv, used, v = lax.while_loop(
                path_cond, path_body, (j0v, pj0v, way, minv, used, v)
            )

            tmp_v[pl.ds(128, 16)] = way

            def unwind_cond(st):
                _p, j0v_ = st
                return jnp.max(j0v_) != 0

            def unwind_body(st):
                p_, j0v_ = st
                j1v = plsc.load_gather(tmp_v, [j0v_ + 128])
                pj1v = plsc.load_gather(ans_v, [j1v])
                p_ = jnp.where(iota == j0v_, pj1v, p_)
                ans_v[...] = p_
                return (p_, j1v)

            p, _ = lax.while_loop(unwind_cond, unwind_body, (p, j0v))
            return (p, way, v)

        p, _, _ = lax.fori_loop(
            1, N_CLS + 1, augment,
            (zeros_i, zeros_i, jnp.zeros((16,), jnp.float32)),
        )

        ans_v[...] = zeros_i
        valid = (p > 0) & (iota >= 1) & (iota <= N_CLS)
        plsc.store_scatter(
            ans_v, [jnp.maximum(p - 1, 0)], iota - 1, mask=valid
        )
        ans = ans_v[...]
        col = jnp.where((iota < tin) & lane_lt10, ans, 0)
        ans_v[...] = col
        pltpu.sync_copy(ans_v, col_hbm.at[b])
        ans_v[...] = jnp.full((16,), tin)
        pltpu.sync_copy(ans_v, tin_hbm.at[b])


_miou_sc = pl.kernel(
    _miou_body,
    out_type=_OUT_TYPE,
    mesh=_mesh,
    compiler_params=pltpu.CompilerParams(needs_layout_passes=False),
    scratch_types=_SCRATCH_TYPES,
)


BLK = 8192


def _bcast_body(col_ref, tin_ref, out_ref, tin_out_ref):
    colt = col_ref[...].T[0:N_CLS, :]
    out_ref[...] = jnp.broadcast_to(colt[:, :, None], (N_CLS, B, BLK))

    @pl.when(pl.program_id(0) == 0)
    def _tin():
        tin_out_ref[...] = tin_ref[...].T[0, :]


_bcast_tc = pl.pallas_call(
    _bcast_body,
    grid=(N // BLK,),
    in_specs=[
        pl.BlockSpec((B, 16), lambda i: (0, 0)),
        pl.BlockSpec((B, 16), lambda i: (0, 0)),
    ],
    out_specs=[
        pl.BlockSpec((N_CLS, B, BLK), lambda i: (0, 0, i)),
        pl.BlockSpec((B,), lambda i: (0,)),
    ],
    out_shape=[
        jax.ShapeDtypeStruct((N_CLS, B, N), jnp.int32),
        jax.ShapeDtypeStruct((B,), jnp.int32),
    ],
)


def kernel(predicts, targets):
    col8, tin8 = _miou_sc(predicts, targets)
    planes, target_inst_num = _bcast_tc(col8, tin8)
    matching_indices = jnp.transpose(planes, (1, 2, 0))
    return (matching_indices, target_inst_num)
